# baseline (speedup 1.0000x reference)
import functools

import jax
import jax.numpy as jnp
from jax import lax
from jax.experimental import pallas as pl
from jax.experimental.pallas import tpu as pltpu
from jax.experimental.pallas import tpu_sc as plsc

_B = 4096
_L = 200
_EMB = 64
_HID = 128
_NCLS = 100

_CA = 96
_CB = 104

_NC = 2
_NS = 16
_NW = _NC * _NS
_BPW = _B // _NW

_mesh = plsc.VectorSubcoreMesh(core_axis_name="c", subcore_axis_name="s")


@functools.partial(
    pl.kernel,
    out_type=jax.ShapeDtypeStruct((_B, _EMB), jnp.float32),
    mesh=_mesh,
    scratch_types=[
        pltpu.VMEM((_BPW, _CA), jnp.int32),
        pltpu.VMEM((_BPW, _CB), jnp.int32),
        pltpu.VMEM((2, _CA, _EMB), jnp.float32),
        pltpu.VMEM((2, _CB, _EMB), jnp.float32),
        pltpu.VMEM((_BPW, _EMB), jnp.float32),
        pltpu.SemaphoreType.DMA,
        pltpu.SemaphoreType.DMA,
    ],
    compiler_params=pltpu.CompilerParams(use_tc_tiling_on_sc=False),
)
def _pooled_sum(texta, textb, embed, out, idxa, idxb, rowsa, rowsb, acc,
                sema, semb):
    wid = lax.axis_index("s") * _NC + lax.axis_index("c")
    base = wid * _BPW

    pltpu.sync_copy(texta.at[pl.ds(base, _BPW)], idxa)
    pltpu.sync_copy(textb.at[pl.ds(base, _BPW)], idxb)

    def start(row, bank):
        pltpu.make_async_copy(
            embed.at[idxa.at[row]], rowsa.at[bank], sema).start()
        pltpu.make_async_copy(
            embed.at[idxb.at[row]], rowsb.at[bank], semb).start()

    def wait(row, bank):
        pltpu.make_async_copy(
            embed.at[idxa.at[row]], rowsa.at[bank], sema).wait()
        pltpu.make_async_copy(
            embed.at[idxb.at[row]], rowsb.at[bank], semb).wait()

    def accum(ref, n, accs):
        def body(j, accs):
            a0, a1, a2, a3 = accs
            for u in range(4):
                r = j * 4 + u
                a0 = a0 + ref[r, pl.ds(0, 16)]
                a1 = a1 + ref[r, pl.ds(16, 16)]
                a2 = a2 + ref[r, pl.ds(32, 16)]
                a3 = a3 + ref[r, pl.ds(48, 16)]
            return (a0, a1, a2, a3)

        return lax.fori_loop(0, n // 4, body, accs)

    start(0, 0)
    start(1, 1)

    zero = jnp.zeros((16,), jnp.float32)

    def outer(i, carry):
        for bank in range(2):
            row = i * 2 + bank
            wait(row, bank)
            accs = (zero, zero, zero, zero)
            accs = accum(rowsa.at[bank], _CA, accs)
            accs = accum(rowsb.at[bank], _CB, accs)
            acc[row, pl.ds(0, 16)] = accs[0]
            acc[row, pl.ds(16, 16)] = accs[1]
            acc[row, pl.ds(32, 16)] = accs[2]
            acc[row, pl.ds(48, 16)] = accs[3]

            @pl.when(row + 2 < _BPW)
            def _():
                start(row + 2, bank)

        return carry

    lax.fori_loop(0, _BPW // 2, outer, 0)

    pltpu.sync_copy(acc, out.at[pl.ds(base, _BPW)])


def _mlp_body(e_ref, w1_ref, b1_ref, w2_ref, b2_ref, o_ref):
    e = e_ref[...] * (1.0 / _L)
    h = jnp.tanh(
        lax.dot_general(e, w1_ref[...], (((1,), (0,)), ((), ())),
                        preferred_element_type=jnp.float32)
        + b1_ref[...])
    o_ref[...] = (
        lax.dot_general(h, w2_ref[...], (((1,), (0,)), ((), ())),
                        preferred_element_type=jnp.float32)
        + b2_ref[...])


_BB = 512


def _mlp(pooled, w1, b1, w2, b2):
    return pl.pallas_call(
        _mlp_body,
        grid=(_B // _BB,),
        in_specs=[
            pl.BlockSpec((_BB, _EMB), lambda i: (i, 0)),
            pl.BlockSpec((_EMB, _HID), lambda i: (0, 0)),
            pl.BlockSpec((1, _HID), lambda i: (0, 0)),
            pl.BlockSpec((_HID, _NCLS), lambda i: (0, 0)),
            pl.BlockSpec((1, _NCLS), lambda i: (0, 0)),
        ],
        out_specs=pl.BlockSpec((_BB, _NCLS), lambda i: (i, 0)),
        out_shape=jax.ShapeDtypeStruct((_B, _NCLS), jnp.float32),
    )(pooled, w1, b1.reshape(1, _HID), w2, b2.reshape(1, _NCLS))


def kernel(text, embed, w1, b1, w2, b2):
    texta = text[:, :_CA]
    textb = text[:, _CA:]
    pooled = _pooled_sum(texta, textb, embed)
    return _mlp(pooled, w1, b1, w2, b2)
